# baseline (device time: 25513 ns/iter reference)
import jax
import jax.numpy as jnp
from jax import lax
from jax.experimental import pallas as pl
from jax.experimental.pallas import tpu as pltpu

N_DEV = 16


def kernel(A, B):
    m, k = A.shape
    k2, n = B.shape
    rows = m // N_DEV

    def body(a_ref, b_ref, out_ref, z16, rs_buf, ag_buf,
             rs_send, rs_recv, ag_send, ag_recv):
        d = lax.axis_index("i")

        def chunk(c):
            return pl.ds(lax.rem(c + 2 * N_DEV, N_DEV) * rows, rows)

        barrier_sem = pltpu.get_barrier_semaphore()
        for off in range(1, N_DEV):
            pl.semaphore_signal(
                barrier_sem, inc=1,
                device_id=(lax.rem(d + off, N_DEV),),
                device_id_type=pl.DeviceIdType.MESH,
            )

        zv = jnp.dot(
            a_ref[:, :], b_ref[:, :], preferred_element_type=jnp.float32
        )
        out_ref[:, :] = zv
        z16[:, :] = zv.astype(jnp.bfloat16)

        pl.semaphore_wait(barrier_sem, N_DEV - 1)

        rs_rdmas = []
        for off in range(N_DEV - 1, 0, -1):
            o = lax.rem(d + off, N_DEV)
            rdma = pltpu.make_async_remote_copy(
                src_ref=z16.at[chunk(o)],
                dst_ref=rs_buf.at[d],
                send_sem=rs_send.at[o],
                recv_sem=rs_recv.at[d],
                device_id=(o,),
                device_id_type=pl.DeviceIdType.MESH,
            )
            rdma.start()
            rs_rdmas.append(rdma)

        acc = out_ref[chunk(d)]
        for off in range(1, N_DEV):
            src = lax.rem(d + off, N_DEV)
            recv = pltpu.make_async_remote_copy(
                src_ref=rs_buf.at[src],
                dst_ref=rs_buf.at[src],
                send_sem=rs_send.at[src],
                recv_sem=rs_recv.at[src],
                device_id=(src,),
                device_id_type=pl.DeviceIdType.MESH,
            )
            recv.wait_recv()
            acc = acc + rs_buf[src].astype(jnp.float32)

        z = acc / (1.0 + jnp.exp(-acc))
        out_ref[chunk(d)] = z
        z16[chunk(d)] = z.astype(jnp.bfloat16)

        ag_rdmas = []
        for off in range(N_DEV - 1, 0, -1):
            o = lax.rem(d + off, N_DEV)
            rdma = pltpu.make_async_remote_copy(
                src_ref=z16.at[chunk(d)],
                dst_ref=ag_buf.at[d],
                send_sem=ag_send.at[o],
                recv_sem=ag_recv.at[d],
                device_id=(o,),
                device_id_type=pl.DeviceIdType.MESH,
            )
            rdma.start()
            ag_rdmas.append(rdma)

        for off in range(1, N_DEV):
            src = lax.rem(d + off, N_DEV)
            recv = pltpu.make_async_remote_copy(
                src_ref=ag_buf.at[src],
                dst_ref=ag_buf.at[src],
                send_sem=ag_send.at[src],
                recv_sem=ag_recv.at[src],
                device_id=(src,),
                device_id_type=pl.DeviceIdType.MESH,
            )
            recv.wait_recv()
            out_ref[chunk(src)] = ag_buf[src].astype(jnp.float32)

        for rdma in rs_rdmas:
            rdma.wait_send()
        for rdma in ag_rdmas:
            rdma.wait_send()

    return pl.pallas_call(
        body,
        out_shape=jax.ShapeDtypeStruct((m, n), jnp.float32),
        in_specs=[
            pl.BlockSpec(memory_space=pltpu.VMEM),
            pl.BlockSpec(memory_space=pltpu.VMEM),
        ],
        out_specs=pl.BlockSpec(memory_space=pltpu.VMEM),
        scratch_shapes=[
            pltpu.VMEM((m, n), jnp.bfloat16),
            pltpu.VMEM((N_DEV, m // N_DEV, n), jnp.bfloat16),
            pltpu.VMEM((N_DEV, m // N_DEV, n), jnp.bfloat16),
            pltpu.SemaphoreType.DMA((N_DEV,)),
            pltpu.SemaphoreType.DMA((N_DEV,)),
            pltpu.SemaphoreType.DMA((N_DEV,)),
            pltpu.SemaphoreType.DMA((N_DEV,)),
        ],
        compiler_params=pltpu.CompilerParams(collective_id=0),
    )(A, B)


# device time: 22422 ns/iter; 1.1379x vs baseline; 1.1379x over previous
import jax
import jax.numpy as jnp
from jax import lax
from jax.experimental import pallas as pl
from jax.experimental.pallas import tpu as pltpu

N_DEV = 16


def kernel(A, B):
    m, k = A.shape
    k2, n = B.shape
    rows = m // N_DEV

    def body(a_ref, b_ref, out_ref, send16, rs_buf, ag_buf,
             rs_send, rs_recv, ag_send, ag_recv):
        d = lax.axis_index("i")

        def chunk(c):
            return pl.ds(lax.rem(c + 2 * N_DEV, N_DEV) * rows, rows)

        barrier_sem = pltpu.get_barrier_semaphore()
        for off in range(1, N_DEV):
            pl.semaphore_signal(
                barrier_sem, inc=1,
                device_id=(lax.rem(d + off, N_DEV),),
                device_id_type=pl.DeviceIdType.MESH,
            )

        out_ref[:, :] = jnp.dot(
            a_ref[:, :], b_ref[:, :], preferred_element_type=jnp.float32
        )

        for off in range(1, N_DEV):
            o = lax.rem(d + off, N_DEV)
            send16[off] = out_ref[chunk(o)].astype(jnp.bfloat16)

        pl.semaphore_wait(barrier_sem, N_DEV - 1)

        rs_rdmas = []
        for off in range(1, N_DEV):
            o = lax.rem(d + off, N_DEV)
            rdma = pltpu.make_async_remote_copy(
                src_ref=send16.at[off],
                dst_ref=rs_buf.at[d],
                send_sem=rs_send.at[o],
                recv_sem=rs_recv.at[d],
                device_id=(o,),
                device_id_type=pl.DeviceIdType.MESH,
            )
            rdma.start()
            rs_rdmas.append(rdma)

        acc = out_ref[chunk(d)]
        for off in range(1, N_DEV):
            src = lax.rem(d + off, N_DEV)
            recv = pltpu.make_async_remote_copy(
                src_ref=rs_buf.at[src],
                dst_ref=rs_buf.at[src],
                send_sem=rs_send.at[src],
                recv_sem=rs_recv.at[src],
                device_id=(src,),
                device_id_type=pl.DeviceIdType.MESH,
            )
            recv.wait_recv()
            acc = acc + rs_buf[src].astype(jnp.float32)

        z = acc / (1.0 + jnp.exp(-acc))
        out_ref[chunk(d)] = z
        send16[0] = z.astype(jnp.bfloat16)

        ag_rdmas = []
        for off in range(1, N_DEV):
            o = lax.rem(d + off, N_DEV)
            rdma = pltpu.make_async_remote_copy(
                src_ref=send16.at[0],
                dst_ref=ag_buf.at[d],
                send_sem=ag_send.at[o],
                recv_sem=ag_recv.at[d],
                device_id=(o,),
                device_id_type=pl.DeviceIdType.MESH,
            )
            rdma.start()
            ag_rdmas.append(rdma)

        for off in range(1, N_DEV):
            src = lax.rem(d + off, N_DEV)
            recv = pltpu.make_async_remote_copy(
                src_ref=ag_buf.at[src],
                dst_ref=ag_buf.at[src],
                send_sem=ag_send.at[src],
                recv_sem=ag_recv.at[src],
                device_id=(src,),
                device_id_type=pl.DeviceIdType.MESH,
            )
            recv.wait_recv()
            out_ref[chunk(src)] = ag_buf[src].astype(jnp.float32)

        for rdma in rs_rdmas:
            rdma.wait_send()
        for rdma in ag_rdmas:
            rdma.wait_send()

    return pl.pallas_call(
        body,
        out_shape=jax.ShapeDtypeStruct((m, n), jnp.float32),
        in_specs=[
            pl.BlockSpec(memory_space=pltpu.VMEM),
            pl.BlockSpec(memory_space=pltpu.VMEM),
        ],
        out_specs=pl.BlockSpec(memory_space=pltpu.VMEM),
        scratch_shapes=[
            pltpu.VMEM((N_DEV, m // N_DEV, n), jnp.bfloat16),
            pltpu.VMEM((N_DEV, m // N_DEV, n), jnp.bfloat16),
            pltpu.VMEM((N_DEV, m // N_DEV, n), jnp.bfloat16),
            pltpu.SemaphoreType.DMA((N_DEV,)),
            pltpu.SemaphoreType.DMA((N_DEV,)),
            pltpu.SemaphoreType.DMA((N_DEV,)),
            pltpu.SemaphoreType.DMA((N_DEV,)),
        ],
        compiler_params=pltpu.CompilerParams(collective_id=0),
    )(A, B)
